# baseline (device time: 10756 ns/iter reference)
import jax
import jax.numpy as jnp
from jax import lax
from jax.experimental import pallas as pl
from jax.experimental.pallas import tpu as pltpu

N_DEV = 4


def kernel(x, t_emb, W_scale, W_shift):
    b, s, c_local = x.shape

    def body(x_ref, t_ref, ws_ref, wsh_ref, out_ref, comm_ref, send_sems, recv_sems):
        my = lax.axis_index("i")

        barrier_sem = pltpu.get_barrier_semaphore()
        for d in range(1, N_DEV):
            pl.semaphore_signal(
                barrier_sem, inc=1,
                device_id=((my + d) % N_DEV,),
                device_id_type=pl.DeviceIdType.MESH,
            )
        pl.semaphore_wait(barrier_sem, N_DEV - 1)

        comm_ref[0, 0] = jnp.zeros((b, s), jnp.float32)
        comm_ref[0, 1] = jnp.zeros((b, s), jnp.float32)

        rdmas = []
        for d in range(1, N_DEV):
            rdma = pltpu.make_async_remote_copy(
                src_ref=comm_ref.at[0],
                dst_ref=comm_ref.at[N_DEV - d],
                send_sem=send_sems.at[d - 1],
                recv_sem=recv_sems.at[d - 1],
                device_id=((my + d) % N_DEV,),
                device_id_type=pl.DeviceIdType.MESH,
            )
            rdma.start()
            rdmas.append(rdma)
        for rdma in rdmas:
            rdma.wait()

        out_ref[...] = x_ref[...] + comm_ref[1, 0, 0, 0]

    return pl.pallas_call(
        body,
        out_shape=jax.ShapeDtypeStruct((b, s, c_local), jnp.float32),
        in_specs=[pl.BlockSpec(memory_space=pltpu.VMEM)] * 4,
        out_specs=pl.BlockSpec(memory_space=pltpu.VMEM),
        scratch_shapes=[
            pltpu.VMEM((N_DEV, 2, b, s), jnp.float32),
            pltpu.SemaphoreType.DMA((N_DEV - 1,)),
            pltpu.SemaphoreType.DMA((N_DEV - 1,)),
        ],
        compiler_params=pltpu.CompilerParams(collective_id=0),
    )(x, t_emb, W_scale, W_shift)


# device time: 9657 ns/iter; 1.1138x vs baseline; 1.1138x over previous
import jax
import jax.numpy as jnp
from jax import lax
from jax.experimental import pallas as pl
from jax.experimental.pallas import tpu as pltpu

N_DEV = 4
EPS = 1e-5
K = 2


def kernel(x, t_emb, W_scale, W_shift):
    b, s, c_local = x.shape
    c_global = c_local * N_DEV
    sc = s // K

    def body(x_ref, t_ref, ws_ref, wsh_ref, out_ref, comm, send_sems, recv_sems):
        my = lax.axis_index("i")

        barrier_sem = pltpu.get_barrier_semaphore()
        for d in range(1, N_DEV):
            pl.semaphore_signal(
                barrier_sem, inc=1,
                device_id=((my + d) % N_DEV,),
                device_id_type=pl.DeviceIdType.MESH,
            )

        xs = x_ref[...]

        scale1 = 1.0 + jnp.dot(t_ref[...], ws_ref[...],
                               preferred_element_type=jnp.float32)
        shift = jnp.dot(t_ref[...], wsh_ref[...],
                        preferred_element_type=jnp.float32)

        rdmas = [[None] * N_DEV for _ in range(K)]
        for k in range(K):
            xk = xs[:, k * sc:(k + 1) * sc, :]
            comm[0, 0, k] = jnp.sum(xk, axis=-1)
            comm[0, 1, k] = jnp.sum(xk * xk, axis=-1)
            if k == 0:
                pl.semaphore_wait(barrier_sem, N_DEV - 1)
            for d in range(1, N_DEV):
                i = (d - 1) * K + k
                rdma = pltpu.make_async_remote_copy(
                    src_ref=comm.at[0, :, k],
                    dst_ref=comm.at[N_DEV - d, :, k],
                    send_sem=send_sems.at[i],
                    recv_sem=recv_sems.at[i],
                    device_id=((my + d) % N_DEV,),
                    device_id_type=pl.DeviceIdType.MESH,
                )
                rdma.start()
                rdmas[k][d] = rdma

        inv_c = 1.0 / c_global

        for k in range(K):
            for d in range(1, N_DEV):
                rdmas[k][d].wait()
            s1 = comm[0, 0, k] + comm[1, 0, k] + comm[2, 0, k] + comm[3, 0, k]
            s2 = comm[0, 1, k] + comm[1, 1, k] + comm[2, 1, k] + comm[3, 1, k]
            mean = s1 * inv_c
            var = s2 * inv_c - mean * mean
            rstd = lax.rsqrt(var + EPS)
            xk = xs[:, k * sc:(k + 1) * sc, :]
            h = (xk - mean[:, :, None]) * rstd[:, :, None]
            out_ref[:, k * sc:(k + 1) * sc, :] = (
                h * scale1[:, None, :] + shift[:, None, :]
            )

    return pl.pallas_call(
        body,
        out_shape=jax.ShapeDtypeStruct((b, s, c_local), jnp.float32),
        in_specs=[
            pl.BlockSpec(memory_space=pltpu.VMEM),
            pl.BlockSpec(memory_space=pltpu.VMEM),
            pl.BlockSpec(memory_space=pltpu.VMEM),
            pl.BlockSpec(memory_space=pltpu.VMEM),
        ],
        out_specs=pl.BlockSpec(memory_space=pltpu.VMEM),
        scratch_shapes=[
            pltpu.VMEM((N_DEV, 2, K, b, sc), jnp.float32),
            pltpu.SemaphoreType.DMA(((N_DEV - 1) * K,)),
            pltpu.SemaphoreType.DMA(((N_DEV - 1) * K,)),
        ],
        input_output_aliases={0: 0},
        compiler_params=pltpu.CompilerParams(collective_id=0),
    )(x, t_emb, W_scale, W_shift)
